# Initial kernel scaffold; baseline (speedup 1.0000x reference)
import jax
import jax.numpy as jnp
from jax.experimental import pallas as pl


def kernel(x, edge_index, W1_l, b1, W1_r, W2_l, b2, W2_r):
    raise NotImplementedError("write your pallas kernel here")


# speedup vs baseline: 7.6933x; 7.6933x over previous
import functools

import jax
import jax.numpy as jnp
from jax import lax
from jax.experimental import pallas as pl
from jax.experimental.pallas import tpu as pltpu
from jax.experimental.pallas import tpu_sc as plsc

N = 100000
E = 1600000
NC = 2
NS = 16
N_PAD = 100096
HALF = N_PAD // 2
D = 16
EPT = E // NS
CHUNK = 800
NCHUNK = EPT // CHUNK
RPT = HALF // NS


@functools.lru_cache(maxsize=None)
def _make_seg_sum():
    mesh = plsc.VectorSubcoreMesh(
        core_axis_name="c", subcore_axis_name="s",
        num_cores=NC, num_subcores=NS)

    @functools.partial(
        pl.kernel,
        mesh=mesh,
        compiler_params=pltpu.CompilerParams(use_tc_tiling_on_sc=False),
        out_type=jax.ShapeDtypeStruct((NC, HALF, D), jnp.float32),
        scratch_types=[
            pltpu.VMEM((CHUNK,), jnp.int32),
            pltpu.VMEM((CHUNK,), jnp.int32),
            pltpu.VMEM((CHUNK, D), jnp.float32),
            pltpu.VMEM((RPT, D), jnp.float32),
            pltpu.VMEM_SHARED((HALF + 8, D), jnp.float32),
            pltpu.SemaphoreType.DMA,
        ],
    )
    def seg_sum(table, src, dst, zeros, out, src_v, dst_v, rows_v,
                stage_v, acc_sh, sem):
        c = lax.axis_index("c")
        s = lax.axis_index("s")
        rbase = s * RPT
        dst_base = c * HALF
        pltpu.sync_copy(zeros, stage_v)
        pltpu.sync_copy(stage_v, acc_sh.at[pl.ds(rbase, RPT)])
        plsc.subcore_barrier()

        def chunk_body(i, _):
            ebase = s * EPT + i * CHUNK
            pltpu.sync_copy(src.at[pl.ds(ebase, CHUNK)], src_v)
            pltpu.sync_copy(dst.at[pl.ds(ebase, CHUNK)], dst_v)
            pltpu.async_copy(table.at[src_v], rows_v, sem).wait()

            def remap(k, _):
                off = k * 16
                d16 = dst_v[pl.ds(off, 16)] - dst_base
                ok = (d16 >= 0) & (d16 < HALF)
                dst_v[pl.ds(off, 16)] = jnp.where(ok, d16, HALF)
                return 0

            lax.fori_loop(0, CHUNK // 16, remap, 0)
            pltpu.sync_copy(rows_v, acc_sh.at[dst_v], add=True)
            return 0

        lax.fori_loop(0, NCHUNK, chunk_body, 0)
        plsc.subcore_barrier()
        pltpu.sync_copy(acc_sh.at[pl.ds(rbase, RPT)], stage_v)
        pltpu.sync_copy(stage_v, out.at[c, pl.ds(rbase, RPT)])

    return seg_sum


def _seg_sum(table16, src, dst):
    zeros = jnp.zeros((RPT, D), jnp.float32)
    acc = _make_seg_sum()(table16, src, dst, zeros)
    return acc.reshape(N_PAD, D)


_BLK = 2048


def _phase2_body(acc_ref, x_ref, w1l_ref, b1_ref, w1r_ref, w2l_ref,
                 w2r_ref, p_ref, q_ref, invc_ref):
    s = acc_ref[...]
    cnt = jnp.maximum(s[:, 2:3], 1.0)
    invc = 1.0 / cnt
    m = s[:, 0:2] * invc
    xb = x_ref[...]
    h = (jnp.dot(m, w1l_ref[...].T, preferred_element_type=jnp.float32)
         + jnp.dot(xb, w1r_ref[...].T, preferred_element_type=jnp.float32)
         + b1_ref[...])
    h = jnp.maximum(h, 0.0)
    p2 = jnp.dot(h, w2l_ref[...].T, preferred_element_type=jnp.float32)
    p_ref[...] = jnp.concatenate(
        [p2, jnp.zeros((p2.shape[0], D - 2), jnp.float32)], axis=1)
    q_ref[...] = jnp.dot(h, w2r_ref[...].T, preferred_element_type=jnp.float32)
    invc_ref[...] = invc


def _phase2(acc1, x, W1_l, b1, W1_r, W2_l, W2_r):
    grid = (pl.cdiv(N, _BLK),)
    return pl.pallas_call(
        _phase2_body,
        grid=grid,
        in_specs=[
            pl.BlockSpec((_BLK, D), lambda i: (i, 0)),
            pl.BlockSpec((_BLK, 2), lambda i: (i, 0)),
            pl.BlockSpec((32, 2), lambda i: (0, 0)),
            pl.BlockSpec((1, 32), lambda i: (0, 0)),
            pl.BlockSpec((32, 2), lambda i: (0, 0)),
            pl.BlockSpec((2, 32), lambda i: (0, 0)),
            pl.BlockSpec((2, 32), lambda i: (0, 0)),
        ],
        out_specs=[
            pl.BlockSpec((_BLK, D), lambda i: (i, 0)),
            pl.BlockSpec((_BLK, 2), lambda i: (i, 0)),
            pl.BlockSpec((_BLK, 1), lambda i: (i, 0)),
        ],
        out_shape=[
            jax.ShapeDtypeStruct((N, D), jnp.float32),
            jax.ShapeDtypeStruct((N, 2), jnp.float32),
            jax.ShapeDtypeStruct((N, 1), jnp.float32),
        ],
    )(acc1, x, W1_l, b1.reshape(1, 32), W1_r, W2_l, W2_r)


def _phase4_body(acc_ref, invc_ref, q_ref, b2_ref, out_ref):
    s = acc_ref[:, 0:2]
    o = s * invc_ref[...] + q_ref[...] + b2_ref[...]
    mx = jnp.max(o, axis=1, keepdims=True)
    lse = mx + jnp.log(jnp.sum(jnp.exp(o - mx), axis=1, keepdims=True))
    out_ref[...] = o - lse


def _phase4(acc2, invc, q, b2):
    grid = (pl.cdiv(N, _BLK),)
    return pl.pallas_call(
        _phase4_body,
        grid=grid,
        in_specs=[
            pl.BlockSpec((_BLK, D), lambda i: (i, 0)),
            pl.BlockSpec((_BLK, 1), lambda i: (i, 0)),
            pl.BlockSpec((_BLK, 2), lambda i: (i, 0)),
            pl.BlockSpec((1, 2), lambda i: (0, 0)),
        ],
        out_specs=pl.BlockSpec((_BLK, 2), lambda i: (i, 0)),
        out_shape=jax.ShapeDtypeStruct((N, 2), jnp.float32),
    )(acc2, invc, q, b2.reshape(1, 2))


def kernel(x, edge_index, W1_l, b1, W1_r, W2_l, b2, W2_r):
    src = edge_index[0]
    dst = edge_index[1]
    table1 = jnp.concatenate(
        [x, jnp.ones((N, 1), jnp.float32),
         jnp.zeros((N, D - 3), jnp.float32)], axis=1)
    acc1 = _seg_sum(table1, src, dst)
    p16, q, invc = _phase2(acc1[:N], x, W1_l, b1, W1_r, W2_l, W2_r)
    acc2 = _seg_sum(p16, src, dst)
    return _phase4(acc2[:N], invc, q, b2)


# speedup vs baseline: 7.7516x; 1.0076x over previous
import functools

import jax
import jax.numpy as jnp
from jax import lax
from jax.experimental import pallas as pl
from jax.experimental.pallas import tpu as pltpu
from jax.experimental.pallas import tpu_sc as plsc

N = 100000
E = 1600000
NC = 2
NS = 16
N_PAD = 100096
HALF = N_PAD // 2
D = 16
EPT = E // NS
CHUNK = 800
NCHUNK = EPT // CHUNK
RPT = HALF // NS
RHLF = RPT // 4


@functools.lru_cache(maxsize=None)
def _make_seg_sum():
    mesh = plsc.VectorSubcoreMesh(
        core_axis_name="c", subcore_axis_name="s",
        num_cores=NC, num_subcores=NS)

    @functools.partial(
        pl.kernel,
        mesh=mesh,
        compiler_params=pltpu.CompilerParams(use_tc_tiling_on_sc=False),
        out_type=jax.ShapeDtypeStruct((NC, HALF, D), jnp.float32),
        scratch_types=[
            pltpu.VMEM((CHUNK,), jnp.int32),
            pltpu.VMEM((CHUNK,), jnp.int32),
            pltpu.VMEM((CHUNK,), jnp.int32),
            pltpu.VMEM((CHUNK,), jnp.int32),
            pltpu.VMEM((CHUNK,), jnp.int32),
            pltpu.VMEM((CHUNK,), jnp.int32),
            pltpu.VMEM((CHUNK, D), jnp.float32),
            pltpu.VMEM((CHUNK, D), jnp.float32),
            pltpu.VMEM((RHLF, D), jnp.float32),
            pltpu.VMEM_SHARED((HALF + 8, D), jnp.float32),
            pltpu.SemaphoreType.DMA,
            pltpu.SemaphoreType.DMA,
            pltpu.SemaphoreType.DMA,
            pltpu.SemaphoreType.DMA,
            pltpu.SemaphoreType.DMA,
            pltpu.SemaphoreType.DMA,
            pltpu.SemaphoreType.DMA,
            pltpu.SemaphoreType.DMA,
        ],
    )
    def seg_sum(table, src, dst, zeros, out,
                src_v0, src_v1, dst_v0, dst_v1, dstm_v0, dstm_v1,
                rows_v0, rows_v1, stage_v, acc_sh,
                ssem0, ssem1, dsem0, dsem1, gsem0, gsem1, psem0, psem1):
        c = lax.axis_index("c")
        s = lax.axis_index("s")
        rbase = s * RPT
        dst_base = c * HALF
        SRC = (src_v0, src_v1)
        DST = (dst_v0, dst_v1)
        DSTM = (dstm_v0, dstm_v1)
        ROWS = (rows_v0, rows_v1)
        SSEM = (ssem0, ssem1)
        DSEM = (dsem0, dsem1)
        GSEM = (gsem0, gsem1)
        PSEM = (psem0, psem1)

        pltpu.sync_copy(zeros, stage_v)
        for j in range(4):
            pltpu.sync_copy(stage_v, acc_sh.at[pl.ds(rbase + j * RHLF, RHLF)])
        plsc.subcore_barrier()

        def issue_idx(i, b):
            eb = s * EPT + jnp.minimum(i, NCHUNK - 1) * CHUNK
            pltpu.async_copy(src.at[pl.ds(eb, CHUNK)], SRC[b], SSEM[b])
            pltpu.async_copy(dst.at[pl.ds(eb, CHUNK)], DST[b], DSEM[b])

        def wait_idx(b):
            pltpu.make_async_copy(src.at[pl.ds(0, CHUNK)], SRC[b], SSEM[b]).wait()
            pltpu.make_async_copy(dst.at[pl.ds(0, CHUNK)], DST[b], DSEM[b]).wait()

        def remap(b):
            def body(k, _):
                off = k * 16
                d16 = DST[b][pl.ds(off, 16)] - dst_base
                ok = (d16 >= 0) & (d16 < HALF)
                DSTM[b][pl.ds(off, 16)] = jnp.where(ok, d16, HALF)
                return 0
            lax.fori_loop(0, CHUNK // 16, body, 0)

        def gather_start(b):
            pltpu.async_copy(table.at[SRC[b]], ROWS[b], GSEM[b])

        def gather_wait(b):
            pltpu.make_async_copy(table.at[SRC[b]], ROWS[b], GSEM[b]).wait()

        def scatter_start(b):
            pltpu.async_copy(ROWS[b], acc_sh.at[DSTM[b]], PSEM[b], add=True)

        def scatter_wait(b):
            pltpu.make_async_copy(ROWS[b], acc_sh.at[DSTM[b]], PSEM[b]).wait()

        def run_chunk(i, b):
            wait_idx(b)
            gather_start(b)
            remap(b)
            gather_wait(b)
            scatter_start(b)
            issue_idx(i + 2, b)

        issue_idx(0, 0)
        issue_idx(1, 1)
        run_chunk(0, 0)
        run_chunk(1, 1)

        def pair_body(j, _):
            for b in range(2):
                scatter_wait(b)
                run_chunk(2 * j + b, b)
            return 0

        lax.fori_loop(1, NCHUNK // 2, pair_body, 0)
        scatter_wait(0)
        run_chunk(NCHUNK - 1, 0)
        for b in range(2):
            scatter_wait(b)
            wait_idx(b)
        plsc.subcore_barrier()
        for j in range(4):
            pltpu.sync_copy(acc_sh.at[pl.ds(rbase + j * RHLF, RHLF)], stage_v)
            pltpu.sync_copy(stage_v, out.at[c, pl.ds(rbase + j * RHLF, RHLF)])

    return seg_sum


def _seg_sum(table16, src, dst):
    zeros = jnp.zeros((RHLF, D), jnp.float32)
    acc = _make_seg_sum()(table16, src, dst, zeros)
    return acc.reshape(N_PAD, D)


_BLK = 2048


def _phase2_body(acc_ref, x_ref, w1l_ref, b1_ref, w1r_ref, w2l_ref,
                 w2r_ref, p_ref, q_ref, invc_ref):
    s = acc_ref[...]
    cnt = jnp.maximum(s[:, 2:3], 1.0)
    invc = 1.0 / cnt
    m = s[:, 0:2] * invc
    xb = x_ref[...]
    h = (jnp.dot(m, w1l_ref[...].T, preferred_element_type=jnp.float32)
         + jnp.dot(xb, w1r_ref[...].T, preferred_element_type=jnp.float32)
         + b1_ref[...])
    h = jnp.maximum(h, 0.0)
    p2 = jnp.dot(h, w2l_ref[...].T, preferred_element_type=jnp.float32)
    p_ref[...] = jnp.concatenate(
        [p2, jnp.zeros((p2.shape[0], D - 2), jnp.float32)], axis=1)
    q_ref[...] = jnp.dot(h, w2r_ref[...].T, preferred_element_type=jnp.float32)
    invc_ref[...] = invc


def _phase2(acc1, x, W1_l, b1, W1_r, W2_l, W2_r):
    grid = (pl.cdiv(N, _BLK),)
    return pl.pallas_call(
        _phase2_body,
        grid=grid,
        in_specs=[
            pl.BlockSpec((_BLK, D), lambda i: (i, 0)),
            pl.BlockSpec((_BLK, 2), lambda i: (i, 0)),
            pl.BlockSpec((32, 2), lambda i: (0, 0)),
            pl.BlockSpec((1, 32), lambda i: (0, 0)),
            pl.BlockSpec((32, 2), lambda i: (0, 0)),
            pl.BlockSpec((2, 32), lambda i: (0, 0)),
            pl.BlockSpec((2, 32), lambda i: (0, 0)),
        ],
        out_specs=[
            pl.BlockSpec((_BLK, D), lambda i: (i, 0)),
            pl.BlockSpec((_BLK, 2), lambda i: (i, 0)),
            pl.BlockSpec((_BLK, 1), lambda i: (i, 0)),
        ],
        out_shape=[
            jax.ShapeDtypeStruct((N, D), jnp.float32),
            jax.ShapeDtypeStruct((N, 2), jnp.float32),
            jax.ShapeDtypeStruct((N, 1), jnp.float32),
        ],
    )(acc1, x, W1_l, b1.reshape(1, 32), W1_r, W2_l, W2_r)


def _phase4_body(acc_ref, invc_ref, q_ref, b2_ref, out_ref):
    s = acc_ref[:, 0:2]
    o = s * invc_ref[...] + q_ref[...] + b2_ref[...]
    mx = jnp.max(o, axis=1, keepdims=True)
    lse = mx + jnp.log(jnp.sum(jnp.exp(o - mx), axis=1, keepdims=True))
    out_ref[...] = o - lse


def _phase4(acc2, invc, q, b2):
    grid = (pl.cdiv(N, _BLK),)
    return pl.pallas_call(
        _phase4_body,
        grid=grid,
        in_specs=[
            pl.BlockSpec((_BLK, D), lambda i: (i, 0)),
            pl.BlockSpec((_BLK, 1), lambda i: (i, 0)),
            pl.BlockSpec((_BLK, 2), lambda i: (i, 0)),
            pl.BlockSpec((1, 2), lambda i: (0, 0)),
        ],
        out_specs=pl.BlockSpec((_BLK, 2), lambda i: (i, 0)),
        out_shape=jax.ShapeDtypeStruct((N, 2), jnp.float32),
    )(acc2, invc, q, b2.reshape(1, 2))


def kernel(x, edge_index, W1_l, b1, W1_r, W2_l, b2, W2_r):
    src = edge_index[0]
    dst = edge_index[1]
    table1 = jnp.concatenate(
        [x, jnp.ones((N, 1), jnp.float32),
         jnp.zeros((N, D - 3), jnp.float32)], axis=1)
    acc1 = _seg_sum(table1, src, dst)
    p16, q, invc = _phase2(acc1[:N], x, W1_l, b1, W1_r, W2_l, W2_r)
    acc2 = _seg_sum(p16, src, dst)
    return _phase4(acc2[:N], invc, q, b2)


# speedup vs baseline: 17.1860x; 2.2171x over previous
import functools

import jax
import jax.numpy as jnp
from jax import lax
from jax.experimental import pallas as pl
from jax.experimental.pallas import tpu as pltpu
from jax.experimental.pallas import tpu_sc as plsc

N = 100000
E = 1600000
NC = 2
NS = 16
N_PAD = 100096
HALF = N_PAD // 2
D = 16
EPT = E // (NC * NS)
CHUNK = 400
NCHUNK = EPT // CHUNK
HROWS = N_PAD // 2
RPT = HROWS // NS
RHLF = RPT // 4


@functools.lru_cache(maxsize=None)
def _make_seg_sum():
    mesh = plsc.VectorSubcoreMesh(
        core_axis_name="c", subcore_axis_name="s",
        num_cores=NC, num_subcores=NS)

    @functools.partial(
        pl.kernel,
        mesh=mesh,
        compiler_params=pltpu.CompilerParams(use_tc_tiling_on_sc=False),
        out_type=jax.ShapeDtypeStruct((NC, HROWS, D), jnp.float32),
        scratch_types=[
            pltpu.VMEM((CHUNK,), jnp.int32),
            pltpu.VMEM((CHUNK,), jnp.int32),
            pltpu.VMEM((CHUNK,), jnp.int32),
            pltpu.VMEM((CHUNK,), jnp.int32),
            pltpu.VMEM((CHUNK,), jnp.int32),
            pltpu.VMEM((CHUNK,), jnp.int32),
            pltpu.VMEM((CHUNK, D), jnp.float32),
            pltpu.VMEM((CHUNK, D), jnp.float32),
            pltpu.VMEM((RHLF, D), jnp.float32),
            pltpu.VMEM_SHARED((HROWS, D), jnp.float32),
            pltpu.SemaphoreType.DMA,
            pltpu.SemaphoreType.DMA,
            pltpu.SemaphoreType.DMA,
            pltpu.SemaphoreType.DMA,
            pltpu.SemaphoreType.DMA,
            pltpu.SemaphoreType.DMA,
            pltpu.SemaphoreType.DMA,
            pltpu.SemaphoreType.DMA,
        ],
    )
    def seg_sum(table, src, dst, zeros, out,
                src_v0, src_v1, dst_v0, dst_v1, dstm_v0, dstm_v1,
                rows_v0, rows_v1, stage_v, acc_sh,
                ssem0, ssem1, dsem0, dsem1, gsem0, gsem1, psem0, psem1):
        c = lax.axis_index("c")
        s = lax.axis_index("s")
        rbase = s * RPT
        SRC = (src_v0, src_v1)
        DST = (dst_v0, dst_v1)
        DSTM = (dstm_v0, dstm_v1)
        ROWS = (rows_v0, rows_v1)
        SSEM = (ssem0, ssem1)
        DSEM = (dsem0, dsem1)
        GSEM = (gsem0, gsem1)
        PSEM = (psem0, psem1)

        pltpu.sync_copy(zeros, stage_v)
        for j in range(4):
            pltpu.sync_copy(stage_v, acc_sh.at[pl.ds(rbase + j * RHLF, RHLF)])
        plsc.subcore_barrier()

        def issue_idx(i, b):
            eb = (c * NS + s) * EPT + jnp.minimum(i, NCHUNK - 1) * CHUNK
            pltpu.async_copy(src.at[pl.ds(eb, CHUNK)], SRC[b], SSEM[b])
            pltpu.async_copy(dst.at[pl.ds(eb, CHUNK)], DST[b], DSEM[b])

        def wait_idx(b):
            pltpu.make_async_copy(src.at[pl.ds(0, CHUNK)], SRC[b], SSEM[b]).wait()
            pltpu.make_async_copy(dst.at[pl.ds(0, CHUNK)], DST[b], DSEM[b]).wait()

        def remap(b):
            def body(k, _):
                off = k * 16
                s16 = SRC[b][pl.ds(off, 16)]
                d16 = DST[b][pl.ds(off, 16)]
                SRC[b][pl.ds(off, 16)] = 2 * s16 + (d16 & 1)
                DSTM[b][pl.ds(off, 16)] = d16 >> 1
                return 0
            lax.fori_loop(0, CHUNK // 16, body, 0)

        def gather_start(b):
            pltpu.async_copy(table.at[SRC[b]], ROWS[b], GSEM[b])

        def gather_wait(b):
            pltpu.make_async_copy(table.at[SRC[b]], ROWS[b], GSEM[b]).wait()

        def scatter_start(b):
            pltpu.async_copy(ROWS[b], acc_sh.at[DSTM[b]], PSEM[b], add=True)

        def scatter_wait(b):
            pltpu.make_async_copy(ROWS[b], acc_sh.at[DSTM[b]], PSEM[b]).wait()

        def run_chunk(i, b):
            wait_idx(b)
            remap(b)
            gather_start(b)
            gather_wait(b)
            scatter_start(b)
            issue_idx(i + 2, b)

        issue_idx(0, 0)
        issue_idx(1, 1)
        run_chunk(0, 0)
        run_chunk(1, 1)

        def pair_body(j, _):
            for b in range(2):
                scatter_wait(b)
                run_chunk(2 * j + b, b)
            return 0

        lax.fori_loop(1, NCHUNK // 2, pair_body, 0)
        scatter_wait(0)
        run_chunk(NCHUNK - 1, 0)
        for b in range(2):
            scatter_wait(b)
            wait_idx(b)
        plsc.subcore_barrier()
        for j in range(4):
            pltpu.sync_copy(acc_sh.at[pl.ds(rbase + j * RHLF, RHLF)], stage_v)
            pltpu.sync_copy(stage_v, out.at[c, pl.ds(rbase + j * RHLF, RHLF)])

    return seg_sum


def _seg_sum(table2, src, dst):
    zeros = jnp.zeros((RHLF, D), jnp.float32)
    acc = _make_seg_sum()(table2, src, dst, zeros)
    accsum = acc[0] + acc[1]
    return accsum[:, :8].reshape(N_PAD, 4)


_BLK = 2048


def _phase2_body(acc_ref, x_ref, w1l_ref, b1_ref, w1r_ref, w2l_ref,
                 w2r_ref, p_ref, q_ref, invc_ref):
    s = acc_ref[...]
    cnt = jnp.maximum(s[:, 2:3], 1.0)
    invc = 1.0 / cnt
    m = s[:, 0:2] * invc
    xb = x_ref[...]
    h = (jnp.dot(m, w1l_ref[...].T, preferred_element_type=jnp.float32)
         + jnp.dot(xb, w1r_ref[...].T, preferred_element_type=jnp.float32)
         + b1_ref[...])
    h = jnp.maximum(h, 0.0)
    p_ref[...] = jnp.dot(h, w2l_ref[...].T, preferred_element_type=jnp.float32)
    q_ref[...] = jnp.dot(h, w2r_ref[...].T, preferred_element_type=jnp.float32)
    invc_ref[...] = invc


def _phase2(acc1, x, W1_l, b1, W1_r, W2_l, W2_r):
    grid = (pl.cdiv(N, _BLK),)
    return pl.pallas_call(
        _phase2_body,
        grid=grid,
        in_specs=[
            pl.BlockSpec((_BLK, 4), lambda i: (i, 0)),
            pl.BlockSpec((_BLK, 2), lambda i: (i, 0)),
            pl.BlockSpec((32, 2), lambda i: (0, 0)),
            pl.BlockSpec((1, 32), lambda i: (0, 0)),
            pl.BlockSpec((32, 2), lambda i: (0, 0)),
            pl.BlockSpec((2, 32), lambda i: (0, 0)),
            pl.BlockSpec((2, 32), lambda i: (0, 0)),
        ],
        out_specs=[
            pl.BlockSpec((_BLK, 2), lambda i: (i, 0)),
            pl.BlockSpec((_BLK, 2), lambda i: (i, 0)),
            pl.BlockSpec((_BLK, 1), lambda i: (i, 0)),
        ],
        out_shape=[
            jax.ShapeDtypeStruct((N, 2), jnp.float32),
            jax.ShapeDtypeStruct((N, 2), jnp.float32),
            jax.ShapeDtypeStruct((N, 1), jnp.float32),
        ],
    )(acc1, x, W1_l, b1.reshape(1, 32), W1_r, W2_l, W2_r)


def _phase4_body(acc_ref, invc_ref, q_ref, b2_ref, out_ref):
    s = acc_ref[:, 0:2]
    o = s * invc_ref[...] + q_ref[...] + b2_ref[...]
    mx = jnp.max(o, axis=1, keepdims=True)
    lse = mx + jnp.log(jnp.sum(jnp.exp(o - mx), axis=1, keepdims=True))
    out_ref[...] = o - lse


def _phase4(acc2, invc, q, b2):
    grid = (pl.cdiv(N, _BLK),)
    return pl.pallas_call(
        _phase4_body,
        grid=grid,
        in_specs=[
            pl.BlockSpec((_BLK, 4), lambda i: (i, 0)),
            pl.BlockSpec((_BLK, 1), lambda i: (i, 0)),
            pl.BlockSpec((_BLK, 2), lambda i: (i, 0)),
            pl.BlockSpec((1, 2), lambda i: (0, 0)),
        ],
        out_specs=pl.BlockSpec((_BLK, 2), lambda i: (i, 0)),
        out_shape=jax.ShapeDtypeStruct((N, 2), jnp.float32),
    )(acc2, invc, q, b2.reshape(1, 2))


def _dual_table(payload3):
    z = jnp.zeros((N, 1), jnp.float32)
    r0 = jnp.concatenate([payload3, jnp.zeros((N, 13), jnp.float32)], axis=1)
    r1 = jnp.concatenate([jnp.zeros((N, 4), jnp.float32), payload3,
                          jnp.zeros((N, 9), jnp.float32)], axis=1)
    return jnp.stack([r0, r1], axis=1).reshape(2 * N, D)


def kernel(x, edge_index, W1_l, b1, W1_r, W2_l, b2, W2_r):
    src = edge_index[0]
    dst = edge_index[1]
    ones = jnp.ones((N, 1), jnp.float32)
    table1 = _dual_table(jnp.concatenate([x, ones], axis=1))
    acc1 = _seg_sum(table1, src, dst)
    p, q, invc = _phase2(acc1[:N], x, W1_l, b1, W1_r, W2_l, W2_r)
    table2 = _dual_table(jnp.concatenate([p, jnp.zeros((N, 1), jnp.float32)],
                                         axis=1))
    acc2 = _seg_sum(table2, src, dst)
    return _phase4(acc2[:N], invc, q, b2)


# speedup vs baseline: 17.1952x; 1.0005x over previous
import functools

import jax
import jax.numpy as jnp
from jax import lax
from jax.experimental import pallas as pl
from jax.experimental.pallas import tpu as pltpu
from jax.experimental.pallas import tpu_sc as plsc

N = 100000
E = 1600000
NC = 2
NS = 16
N_PAD = 100096
D = 16
EPT = E // (NC * NS)
CHUNK = 400
NCHUNK = EPT // CHUNK
HROWS = N_PAD // 2
RPT = HROWS // NS
RHLF = RPT // 4


@functools.lru_cache(maxsize=None)
def _make_seg_sum():
    mesh = plsc.VectorSubcoreMesh(
        core_axis_name="c", subcore_axis_name="s",
        num_cores=NC, num_subcores=NS)

    @functools.partial(
        pl.kernel,
        mesh=mesh,
        compiler_params=pltpu.CompilerParams(use_tc_tiling_on_sc=False),
        out_type=jax.ShapeDtypeStruct((NC, HROWS, D), jnp.float32),
        scratch_types=[
            pltpu.VMEM((CHUNK,), jnp.int32),
            pltpu.VMEM((CHUNK,), jnp.int32),
            pltpu.VMEM((CHUNK,), jnp.int32),
            pltpu.VMEM((CHUNK,), jnp.int32),
            pltpu.VMEM((CHUNK,), jnp.int32),
            pltpu.VMEM((CHUNK,), jnp.int32),
            pltpu.VMEM((CHUNK, D), jnp.float32),
            pltpu.VMEM((CHUNK, D), jnp.float32),
            pltpu.VMEM((RHLF, D), jnp.float32),
            pltpu.VMEM_SHARED((HROWS, D), jnp.float32),
            pltpu.SemaphoreType.DMA,
            pltpu.SemaphoreType.DMA,
            pltpu.SemaphoreType.DMA,
            pltpu.SemaphoreType.DMA,
            pltpu.SemaphoreType.DMA,
            pltpu.SemaphoreType.DMA,
            pltpu.SemaphoreType.DMA,
            pltpu.SemaphoreType.DMA,
        ],
    )
    def seg_sum(table, src, dst, zeros, out,
                src_v0, src_v1, dst_v0, dst_v1, dstm_v0, dstm_v1,
                rows_v0, rows_v1, stage_v, acc_sh,
                ssem0, ssem1, dsem0, dsem1, gsem0, gsem1, psem0, psem1):
        c = lax.axis_index("c")
        s = lax.axis_index("s")
        rbase = s * RPT
        SRC = (src_v0, src_v1)
        DST = (dst_v0, dst_v1)
        DSTM = (dstm_v0, dstm_v1)
        ROWS = (rows_v0, rows_v1)
        SSEM = (ssem0, ssem1)
        DSEM = (dsem0, dsem1)
        GSEM = (gsem0, gsem1)
        PSEM = (psem0, psem1)

        pltpu.sync_copy(zeros, stage_v)
        for j in range(4):
            pltpu.sync_copy(stage_v, acc_sh.at[pl.ds(rbase + j * RHLF, RHLF)])
        plsc.subcore_barrier()

        def issue_idx(i, b):
            eb = (c * NS + s) * EPT + jnp.minimum(i, NCHUNK - 1) * CHUNK
            pltpu.async_copy(src.at[pl.ds(eb, CHUNK)], SRC[b], SSEM[b])
            pltpu.async_copy(dst.at[pl.ds(eb, CHUNK)], DST[b], DSEM[b])

        def wait_idx(b):
            pltpu.make_async_copy(src.at[pl.ds(0, CHUNK)], SRC[b], SSEM[b]).wait()
            pltpu.make_async_copy(dst.at[pl.ds(0, CHUNK)], DST[b], DSEM[b]).wait()

        def remap(b):
            def body(k, _):
                off = k * 16
                s16 = SRC[b][pl.ds(off, 16)]
                d16 = DST[b][pl.ds(off, 16)]
                SRC[b][pl.ds(off, 16)] = 2 * s16 + (d16 & 1)
                DSTM[b][pl.ds(off, 16)] = d16 >> 1
                return 0
            lax.fori_loop(0, CHUNK // 16, body, 0)

        def gather_start(b):
            pltpu.async_copy(table.at[SRC[b]], ROWS[b], GSEM[b])

        def gather_wait(b):
            pltpu.make_async_copy(table.at[SRC[b]], ROWS[b], GSEM[b]).wait()

        def scatter_start(b):
            pltpu.async_copy(ROWS[b], acc_sh.at[DSTM[b]], PSEM[b], add=True)

        def scatter_wait(b):
            pltpu.make_async_copy(ROWS[b], acc_sh.at[DSTM[b]], PSEM[b]).wait()

        def run_chunk(i, b):
            wait_idx(b)
            remap(b)
            gather_start(b)
            gather_wait(b)
            scatter_start(b)
            issue_idx(i + 2, b)

        issue_idx(0, 0)
        issue_idx(1, 1)
        run_chunk(0, 0)
        run_chunk(1, 1)

        def pair_body(j, _):
            for b in range(2):
                scatter_wait(b)
                run_chunk(2 * j + b, b)
            return 0

        lax.fori_loop(1, NCHUNK // 2, pair_body, 0)
        scatter_wait(0)
        run_chunk(NCHUNK - 1, 0)
        for b in range(2):
            scatter_wait(b)
            wait_idx(b)
        plsc.subcore_barrier()
        for j in range(4):
            pltpu.sync_copy(acc_sh.at[pl.ds(rbase + j * RHLF, RHLF)], stage_v)
            pltpu.sync_copy(stage_v, out.at[c, pl.ds(rbase + j * RHLF, RHLF)])

    return seg_sum


def _seg_sum(table2, src, dst):
    zeros = jnp.zeros((RHLF, D), jnp.float32)
    acc = _make_seg_sum()(table2, src, dst, zeros)
    accsum = acc[0] + acc[1]
    return accsum[:, :8].reshape(N_PAD, 4)


_BLK = 2048


def _phase2_body(acc_ref, x_ref, w1l_ref, b1_ref, w1r_ref, w2l_ref,
                 w2r_ref, p_ref, q_ref, invc_ref):
    s = acc_ref[...]
    cnt = jnp.maximum(s[:, 2:3], 1.0)
    invc = 1.0 / cnt
    m = s[:, 0:2] * invc
    xb = x_ref[...]
    h = (jnp.dot(m, w1l_ref[...].T, preferred_element_type=jnp.float32)
         + jnp.dot(xb, w1r_ref[...].T, preferred_element_type=jnp.float32)
         + b1_ref[...])
    h = jnp.maximum(h, 0.0)
    p_ref[...] = jnp.dot(h, w2l_ref[...].T, preferred_element_type=jnp.float32)
    q_ref[...] = jnp.dot(h, w2r_ref[...].T, preferred_element_type=jnp.float32)
    invc_ref[...] = invc


def _phase2(acc1, x, W1_l, b1, W1_r, W2_l, W2_r):
    grid = (pl.cdiv(N, _BLK),)
    return pl.pallas_call(
        _phase2_body,
        grid=grid,
        in_specs=[
            pl.BlockSpec((_BLK, 4), lambda i: (i, 0)),
            pl.BlockSpec((_BLK, 2), lambda i: (i, 0)),
            pl.BlockSpec((32, 2), lambda i: (0, 0)),
            pl.BlockSpec((1, 32), lambda i: (0, 0)),
            pl.BlockSpec((32, 2), lambda i: (0, 0)),
            pl.BlockSpec((2, 32), lambda i: (0, 0)),
            pl.BlockSpec((2, 32), lambda i: (0, 0)),
        ],
        out_specs=[
            pl.BlockSpec((_BLK, 2), lambda i: (i, 0)),
            pl.BlockSpec((_BLK, 2), lambda i: (i, 0)),
            pl.BlockSpec((_BLK, 1), lambda i: (i, 0)),
        ],
        out_shape=[
            jax.ShapeDtypeStruct((N, 2), jnp.float32),
            jax.ShapeDtypeStruct((N, 2), jnp.float32),
            jax.ShapeDtypeStruct((N, 1), jnp.float32),
        ],
    )(acc1, x, W1_l, b1.reshape(1, 32), W1_r, W2_l, W2_r)


def _phase4_body(acc_ref, invc_ref, q_ref, b2_ref, out_ref):
    s = acc_ref[:, 0:2]
    o = s * invc_ref[...] + q_ref[...] + b2_ref[...]
    mx = jnp.max(o, axis=1, keepdims=True)
    lse = mx + jnp.log(jnp.sum(jnp.exp(o - mx), axis=1, keepdims=True))
    out_ref[...] = o - lse


def _phase4(acc2, invc, q, b2):
    grid = (pl.cdiv(N, _BLK),)
    return pl.pallas_call(
        _phase4_body,
        grid=grid,
        in_specs=[
            pl.BlockSpec((_BLK, 4), lambda i: (i, 0)),
            pl.BlockSpec((_BLK, 1), lambda i: (i, 0)),
            pl.BlockSpec((_BLK, 2), lambda i: (i, 0)),
            pl.BlockSpec((1, 2), lambda i: (0, 0)),
        ],
        out_specs=pl.BlockSpec((_BLK, 2), lambda i: (i, 0)),
        out_shape=jax.ShapeDtypeStruct((N, 2), jnp.float32),
    )(acc2, invc, q, b2.reshape(1, 2))


def _dual_table(payload3):
    r0 = jnp.concatenate([payload3, jnp.zeros((N, 13), jnp.float32)], axis=1)
    r1 = jnp.concatenate([jnp.zeros((N, 4), jnp.float32), payload3,
                          jnp.zeros((N, 9), jnp.float32)], axis=1)
    return jnp.stack([r0, r1], axis=1).reshape(2 * N, D)


def kernel(x, edge_index, W1_l, b1, W1_r, W2_l, b2, W2_r):
    src = edge_index[0]
    dst = edge_index[1]
    ones = jnp.ones((N, 1), jnp.float32)
    table1 = _dual_table(jnp.concatenate([x, ones], axis=1))
    acc1 = _seg_sum(table1, src, dst)
    p, q, invc = _phase2(acc1[:N], x, W1_l, b1, W1_r, W2_l, W2_r)
    table2 = _dual_table(jnp.concatenate([p, jnp.zeros((N, 1), jnp.float32)],
                                         axis=1))
    acc2 = _seg_sum(table2, src, dst)
    return _phase4(acc2[:N], invc, q, b2)
